# Initial kernel scaffold; baseline (speedup 1.0000x reference)
import jax
import jax.numpy as jnp
from jax.experimental import pallas as pl


def kernel(x, pos, edge_index, edge_attr, Wk1, bk1, Wk2, bk2, Wk_lin, Wv1, bv1, Wv2, bv2, Wv_lin, Wq, Wskip, Wn1, bn1, Wn2, bn2, Wc1, bc1, Wc2, bc2, Wc_lin, Wself):
    raise NotImplementedError("write your pallas kernel here")


# speedup vs baseline: 4.5748x; 4.5748x over previous
import functools
import math

import jax
import jax.numpy as jnp
from jax import lax
from jax.experimental import pallas as pl
from jax.experimental.pallas import tpu as pltpu
from jax.experimental.pallas import tpu_sc as plsc

N_NODES = 10000
E_EDGES = 320000
NC, NS = 2, 16
NW = NC * NS
EPW = E_EDGES // NW
CH = 80
NCHUNK = EPW // CH
ROWS_PT = N_NODES // NS

F32 = jnp.float32


def _sc_mesh():
    return plsc.VectorSubcoreMesh(
        core_axis_name="c", subcore_axis_name="s", num_cores=NC, num_subcores=NS
    )


def _sc_gather_body(npack, srci, dsti, sg_o, dg_o, siv, div, srows, drows,
                    sem0, sem1):
    wid = lax.axis_index("s") * NC + lax.axis_index("c")

    @pl.loop(0, NCHUNK)
    def _chunk(i):
        base = wid * EPW + i * CH
        pltpu.sync_copy(srci.at[pl.ds(base, CH)], siv)
        pltpu.sync_copy(dsti.at[pl.ds(base, CH)], div)
        cps = pltpu.async_copy(npack.at[siv], srows, sem0)
        cpd = pltpu.async_copy(npack.at[div], drows, sem1)
        cps.wait()
        cpd.wait()
        pltpu.sync_copy(srows, sg_o.at[pl.ds(base, CH)])
        pltpu.sync_copy(drows, dg_o.at[pl.ds(base, CH)])


_sc_gather = pl.kernel(
    _sc_gather_body,
    out_type=(
        jax.ShapeDtypeStruct((E_EDGES, 16), F32),
        jax.ShapeDtypeStruct((E_EDGES, 16), F32),
    ),
    mesh=_sc_mesh(),
    compiler_params=pltpu.CompilerParams(use_tc_tiling_on_sc=False),
    scratch_types=[
        pltpu.VMEM((CH,), jnp.int32),
        pltpu.VMEM((CH,), jnp.int32),
        pltpu.VMEM((CH, 16), F32),
        pltpu.VMEM((CH, 16), F32),
        pltpu.SemaphoreType.DMA,
        pltpu.SemaphoreType.DMA,
    ],
)


BE = 3200
GE = E_EDGES // BE

_RSQRT_F = float(1.0 / math.sqrt(32.0))

_mm = functools.partial(jnp.matmul, precision=jax.lax.Precision.HIGHEST)


def _tc_edge_body(sg, dg, attr, w1cat, b1cat, w2kv, b2kv, wc2p, bc2,
                  wklvl, wqp, w64_o, c_o):
    sgv = sg[...]
    dgv = dg[...]
    d = dgv - sgv
    d3 = d[:, 0:3]
    r2 = jnp.sum(d3 * d3, axis=1, keepdims=True) + 1e-8
    r = jnp.sqrt(r2)
    at = attr[...]
    ein = jnp.concatenate([r, at, jnp.zeros((BE, 3), F32)], axis=1)
    H = jnp.maximum(ein @ w1cat[...] + b1cat[...], 0.0)
    G = H @ w2kv[...] + b2kv[...]
    c_o[...] = H @ wc2p[...] + bc2[...]
    M = sgv @ wklvl[...]
    kq = dgv @ wqp[...]
    krad = G[:, 0:32]
    vrad = G[:, 32:64]
    kl = M[:, 0:32]
    vl = M[:, 32:64]
    logits = jnp.sum(krad * kl * kq, axis=1, keepdims=True) * _RSQRT_F
    ex = jnp.exp(logits)
    w = ex * (vrad * vl)
    w64_o[...] = jnp.concatenate([w, ex, jnp.zeros((BE, 31), F32)], axis=1)


def _full(shape):
    rank = len(shape)
    return pl.BlockSpec(shape, lambda i, _r=rank: (0,) * _r)


_tc_edge = pl.pallas_call(
    _tc_edge_body,
    grid=(GE,),
    in_specs=[
        pl.BlockSpec((BE, 16), lambda i: (i, 0)),
        pl.BlockSpec((BE, 16), lambda i: (i, 0)),
        pl.BlockSpec((BE, 12), lambda i: (i, 0)),
        _full((16, 256)), _full((1, 256)),
        _full((256, 64)), _full((1, 64)),
        _full((256, 128)), _full((1, 128)),
        _full((16, 64)), _full((16, 32)),
    ],
    out_specs=[
        pl.BlockSpec((BE, 64), lambda i: (i, 0)),
        pl.BlockSpec((BE, 128), lambda i: (i, 0)),
    ],
    out_shape=[
        jax.ShapeDtypeStruct((E_EDGES, 64), F32),
        jax.ShapeDtypeStruct((E_EDGES, 128), F32),
    ],
)


def _sc_attn_body(w48, dsti, zrows, accp_o, dv, wv, acc):
    cid = lax.axis_index("c")
    sid = lax.axis_index("s")
    wid = sid * NC + cid
    row0 = sid * ROWS_PT
    pltpu.sync_copy(zrows.at[pl.ds(row0, ROWS_PT)],
                    acc.at[pl.ds(row0, ROWS_PT)])
    plsc.subcore_barrier()

    @pl.loop(0, NCHUNK)
    def _chunk(i):
        base = wid * EPW + i * CH
        pltpu.sync_copy(dsti.at[pl.ds(base, CH)], dv)
        pltpu.sync_copy(w48.at[pl.ds(base, CH)], wv)
        pltpu.sync_copy(wv, acc.at[dv], add=True)

    plsc.subcore_barrier()
    pltpu.sync_copy(acc.at[pl.ds(row0, ROWS_PT)],
                    accp_o.at[cid, pl.ds(row0, ROWS_PT)])


_sc_attn = pl.kernel(
    _sc_attn_body,
    out_type=jax.ShapeDtypeStruct((NC, N_NODES, 64), F32),
    mesh=_sc_mesh(),
    compiler_params=pltpu.CompilerParams(use_tc_tiling_on_sc=False),
    scratch_types=[
        pltpu.VMEM((CH,), jnp.int32),
        pltpu.VMEM((CH, 64), F32),
        pltpu.VMEM_SHARED((N_NODES, 64), F32),
    ],
)


BN = 2000
GN = N_NODES // BN


def _tc_node_body(accp, xin, wskip, wn1, bn1, wn2, bn2, wclin, wself,
                  selw, seld, hc_o, fs_o):
    acc = accp[0] + accp[1]
    attn_num = _mm(acc, selw[...])
    den = _mm(acc, seld[...])
    attn = attn_num / (den + 1e-9)
    h = attn + xin[...] @ wskip[...]
    nrm = jnp.abs(h)
    ph = h / (nrm + 1e-8)
    s1 = jnp.maximum(nrm @ wn1[...] + bn1[...], 0.0)
    s2 = jnp.maximum(s1 @ wn2[...] + bn2[...], 0.0)
    h2 = s2 * ph
    hc_o[...] = h2 @ wclin[...]
    fs_o[...] = h2 @ wself[...]


_tc_node = pl.pallas_call(
    _tc_node_body,
    grid=(GN,),
    in_specs=[
        pl.BlockSpec((NC, BN, 64), lambda i: (0, i, 0)),
        pl.BlockSpec((BN, 6), lambda i: (i, 0)),
        _full((6, 32)),
        _full((32, 32)), _full((1, 32)), _full((32, 32)), _full((1, 32)),
        _full((32, 128)), _full((32, 128)),
        _full((64, 32)), _full((64, 1)),
    ],
    out_specs=[
        pl.BlockSpec((BN, 128), lambda i: (i, 0)),
        pl.BlockSpec((BN, 128), lambda i: (i, 0)),
    ],
    out_shape=[
        jax.ShapeDtypeStruct((N_NODES, 128), F32),
        jax.ShapeDtypeStruct((N_NODES, 128), F32),
    ],
)


def _sc_conv_body(cmat, hcmat, srci, dsti, zrows, aggp_o,
                  sv, dv, cv, gv, agg, sem):
    cid = lax.axis_index("c")
    sid = lax.axis_index("s")
    wid = sid * NC + cid
    row0 = sid * ROWS_PT
    pltpu.sync_copy(zrows.at[pl.ds(row0, ROWS_PT)],
                    agg.at[pl.ds(row0, ROWS_PT)])
    plsc.subcore_barrier()

    @pl.loop(0, NCHUNK)
    def _chunk(i):
        base = wid * EPW + i * CH
        pltpu.sync_copy(srci.at[pl.ds(base, CH)], sv)
        pltpu.sync_copy(dsti.at[pl.ds(base, CH)], dv)
        cp = pltpu.async_copy(hcmat.at[sv], gv, sem)
        pltpu.sync_copy(cmat.at[pl.ds(base, CH)], cv)
        cp.wait()

        @pl.loop(0, CH)
        def _row(rr):
            for j in range(8):
                o = j * 16
                gv[rr, pl.ds(o, 16)] = (
                    gv[rr, pl.ds(o, 16)] * cv[rr, pl.ds(o, 16)])

        pltpu.sync_copy(gv, agg.at[dv], add=True)

    plsc.subcore_barrier()
    pltpu.sync_copy(agg.at[pl.ds(row0, ROWS_PT)],
                    aggp_o.at[cid, pl.ds(row0, ROWS_PT)])


_sc_conv = pl.kernel(
    _sc_conv_body,
    out_type=jax.ShapeDtypeStruct((NC, N_NODES, 128), F32),
    mesh=_sc_mesh(),
    compiler_params=pltpu.CompilerParams(use_tc_tiling_on_sc=False),
    scratch_types=[
        pltpu.VMEM((CH,), jnp.int32),
        pltpu.VMEM((CH,), jnp.int32),
        pltpu.VMEM((CH, 128), F32),
        pltpu.VMEM((CH, 128), F32),
        pltpu.VMEM_SHARED((N_NODES, 128), F32),
        pltpu.SemaphoreType.DMA,
    ],
)


def _tc_merge_body(aggp, fs, out_o):
    out_o[...] = aggp[0] + aggp[1] + fs[...]


_tc_merge = pl.pallas_call(
    _tc_merge_body,
    grid=(GN,),
    in_specs=[
        pl.BlockSpec((NC, BN, 128), lambda i: (0, i, 0)),
        pl.BlockSpec((BN, 128), lambda i: (i, 0)),
    ],
    out_specs=pl.BlockSpec((BN, 128), lambda i: (i, 0)),
    out_shape=jax.ShapeDtypeStruct((N_NODES, 128), F32),
)


def kernel(x, pos, edge_index, edge_attr,
           Wk1, bk1, Wk2, bk2, Wk_lin,
           Wv1, bv1, Wv2, bv2, Wv_lin,
           Wq, Wskip,
           Wn1, bn1, Wn2, bn2,
           Wc1, bc1, Wc2, bc2, Wc_lin, Wself):
    src = edge_index[0]
    dst = edge_index[1]
    npack = jnp.concatenate([pos, x, jnp.zeros((N_NODES, 7), F32)], axis=1)

    sg, dg = _sc_gather(npack, src, dst)

    def packed(w):
        return jnp.zeros((16, w.shape[1]), F32).at[3:9, :].set(w)

    def pad16(w):
        return jnp.concatenate([w, jnp.zeros((3, w.shape[1]), F32)], axis=0)

    z = jnp.zeros
    w1cat = jnp.concatenate([pad16(Wk1), pad16(Wv1), pad16(Wc1)], axis=1)
    b1cat = jnp.concatenate([bk1, bv1, bc1])[None]
    w2kv = jnp.concatenate([
        jnp.concatenate([Wk2, z((64, 32), F32)], axis=1),
        jnp.concatenate([z((64, 32), F32), Wv2], axis=1),
        z((128, 64), F32),
    ], axis=0)
    b2kv = jnp.concatenate([bk2, bv2])[None]
    wc2p = jnp.concatenate([z((128, 128), F32), Wc2], axis=0)
    wklvl = jnp.concatenate([packed(Wk_lin), packed(Wv_lin)], axis=1)

    selw = jnp.zeros((64, 32), F32).at[jnp.arange(32), jnp.arange(32)].set(1.0)
    seld = jnp.zeros((64, 1), F32).at[32, 0].set(1.0)

    w64, c = _tc_edge(
        sg, dg, edge_attr,
        w1cat, b1cat, w2kv, b2kv, wc2p, bc2[None], wklvl, packed(Wq),
    )

    z64 = jnp.zeros((N_NODES, 64), F32)
    accp = _sc_attn(w64, dst, z64)

    hc, fs = _tc_node(
        accp, x, Wskip, Wn1, bn1[None], Wn2, bn2[None], Wc_lin, Wself,
        selw, seld,
    )

    z128 = jnp.zeros((N_NODES, 128), F32)
    aggp = _sc_conv(c, hc, src, dst, z128)

    feat = _tc_merge(aggp, fs)
    return (feat[None], pos)


# speedup vs baseline: 4.9762x; 1.0877x over previous
import functools
import math

import jax
import jax.numpy as jnp
from jax import lax
from jax.experimental import pallas as pl
from jax.experimental.pallas import tpu as pltpu
from jax.experimental.pallas import tpu_sc as plsc

N_NODES = 10000
E_EDGES = 320000
NC, NS = 2, 16
NW = NC * NS
EPW = E_EDGES // NW
CH = 80
NCHUNK = EPW // CH
ROWS_PT = N_NODES // NS

F32 = jnp.float32


def _sc_mesh():
    return plsc.VectorSubcoreMesh(
        core_axis_name="c", subcore_axis_name="s", num_cores=NC, num_subcores=NS
    )


def _sc_gather_body(npack, srci, dsti, sg_o, dg_o,
                    siv0, div0, srows0, drows0, siv1, div1, srows1, drows1,
                    sa0, sb0, sa1, sb1):
    wid = lax.axis_index("s") * NC + lax.axis_index("c")
    bufs = ((siv0, div0, srows0, drows0, sa0, sb0),
            (siv1, div1, srows1, drows1, sa1, sb1))

    def start(i, b):
        siv, div, srows, drows, sa, sb = bufs[b]
        base = wid * EPW + i * CH
        pltpu.sync_copy(srci.at[pl.ds(base, CH)], siv)
        pltpu.sync_copy(dsti.at[pl.ds(base, CH)], div)
        pltpu.async_copy(npack.at[siv], srows, sa)
        pltpu.async_copy(npack.at[div], drows, sb)

    def finish(i, b):
        siv, div, srows, drows, sa, sb = bufs[b]
        base = wid * EPW + i * CH
        pltpu.make_async_copy(npack.at[siv], srows, sa).wait()
        pltpu.make_async_copy(npack.at[div], drows, sb).wait()
        pltpu.sync_copy(srows, sg_o.at[pl.ds(base, CH)])
        pltpu.sync_copy(drows, dg_o.at[pl.ds(base, CH)])

    start(0, 0)

    @pl.loop(0, (NCHUNK - 1) // 2)
    def _pair(p):
        i0 = 2 * p
        start(i0 + 1, 1)
        finish(i0, 0)
        start(i0 + 2, 0)
        finish(i0 + 1, 1)

    finish(NCHUNK - 1, 0)


_sc_gather = pl.kernel(
    _sc_gather_body,
    out_type=(
        jax.ShapeDtypeStruct((E_EDGES, 16), F32),
        jax.ShapeDtypeStruct((E_EDGES, 16), F32),
    ),
    mesh=_sc_mesh(),
    compiler_params=pltpu.CompilerParams(use_tc_tiling_on_sc=False),
    scratch_types=[
        pltpu.VMEM((CH,), jnp.int32),
        pltpu.VMEM((CH,), jnp.int32),
        pltpu.VMEM((CH, 16), F32),
        pltpu.VMEM((CH, 16), F32),
        pltpu.VMEM((CH,), jnp.int32),
        pltpu.VMEM((CH,), jnp.int32),
        pltpu.VMEM((CH, 16), F32),
        pltpu.VMEM((CH, 16), F32),
        pltpu.SemaphoreType.DMA,
        pltpu.SemaphoreType.DMA,
        pltpu.SemaphoreType.DMA,
        pltpu.SemaphoreType.DMA,
    ],
)


BE = 2000
GE = E_EDGES // BE

_RSQRT_F = float(1.0 / math.sqrt(32.0))

_mm = functools.partial(jnp.matmul, precision=jax.lax.Precision.HIGHEST)


def _tc_edge_body(sg, dg, attr, w1cat, b1cat, w2kv, b2kv, wc2p, bc2,
                  wklvl, wqp, w64_o, c_o):
    sgv = sg[...]
    dgv = dg[...]
    d = dgv - sgv
    d3 = d[:, 0:3]
    r2 = jnp.sum(d3 * d3, axis=1, keepdims=True) + 1e-8
    r = jnp.sqrt(r2)
    at = attr[...]
    ein = jnp.concatenate([r, at, jnp.zeros((BE, 3), F32)], axis=1)
    H = jnp.maximum(ein @ w1cat[...] + b1cat[...], 0.0)
    G = H @ w2kv[...] + b2kv[...]
    c_o[...] = H @ wc2p[...] + bc2[...]
    M = sgv @ wklvl[...]
    kq = dgv @ wqp[...]
    krad = G[:, 0:32]
    vrad = G[:, 32:64]
    kl = M[:, 0:32]
    vl = M[:, 32:64]
    logits = jnp.sum(krad * kl * kq, axis=1, keepdims=True) * _RSQRT_F
    ex = jnp.exp(logits)
    w = ex * (vrad * vl)
    w64_o[...] = jnp.concatenate([w, ex, jnp.zeros((BE, 31), F32)], axis=1)


def _full(shape):
    rank = len(shape)
    return pl.BlockSpec(shape, lambda i, _r=rank: (0,) * _r)


_tc_edge = pl.pallas_call(
    _tc_edge_body,
    grid=(GE,),
    in_specs=[
        pl.BlockSpec((BE, 16), lambda i: (i, 0)),
        pl.BlockSpec((BE, 16), lambda i: (i, 0)),
        pl.BlockSpec((BE, 12), lambda i: (i, 0)),
        _full((16, 256)), _full((1, 256)),
        _full((256, 64)), _full((1, 64)),
        _full((256, 128)), _full((1, 128)),
        _full((16, 64)), _full((16, 32)),
    ],
    out_specs=[
        pl.BlockSpec((BE, 64), lambda i: (i, 0)),
        pl.BlockSpec((BE, 128), lambda i: (i, 0)),
    ],
    out_shape=[
        jax.ShapeDtypeStruct((E_EDGES, 64), F32),
        jax.ShapeDtypeStruct((E_EDGES, 128), F32),
    ],
)


def _sc_attn_body(w64, dsti, zrows, accp_o, dv0, wv0, dv1, wv1, sa0, sa1,
                  acc):
    cid = lax.axis_index("c")
    sid = lax.axis_index("s")
    wid = sid * NC + cid
    row0 = sid * ROWS_PT
    pltpu.sync_copy(zrows.at[pl.ds(row0, ROWS_PT)],
                    acc.at[pl.ds(row0, ROWS_PT)])
    bufs = ((dv0, wv0, sa0), (dv1, wv1, sa1))

    def start(i, b):
        dv, wv, sa = bufs[b]
        base = wid * EPW + i * CH
        pltpu.sync_copy(dsti.at[pl.ds(base, CH)], dv)
        pltpu.async_copy(w64.at[pl.ds(base, CH)], wv, sa)

    def finish(i, b):
        dv, wv, sa = bufs[b]
        base = wid * EPW + i * CH
        pltpu.make_async_copy(w64.at[pl.ds(base, CH)], wv, sa).wait()
        pltpu.sync_copy(wv, acc.at[dv], add=True)

    plsc.subcore_barrier()
    start(0, 0)

    @pl.loop(0, (NCHUNK - 1) // 2)
    def _pair(p):
        i0 = 2 * p
        start(i0 + 1, 1)
        finish(i0, 0)
        start(i0 + 2, 0)
        finish(i0 + 1, 1)

    finish(NCHUNK - 1, 0)
    plsc.subcore_barrier()
    pltpu.sync_copy(acc.at[pl.ds(row0, ROWS_PT)],
                    accp_o.at[cid, pl.ds(row0, ROWS_PT)])


_sc_attn = pl.kernel(
    _sc_attn_body,
    out_type=jax.ShapeDtypeStruct((NC, N_NODES, 64), F32),
    mesh=_sc_mesh(),
    compiler_params=pltpu.CompilerParams(use_tc_tiling_on_sc=False),
    scratch_types=[
        pltpu.VMEM((CH,), jnp.int32),
        pltpu.VMEM((CH, 64), F32),
        pltpu.VMEM((CH,), jnp.int32),
        pltpu.VMEM((CH, 64), F32),
        pltpu.SemaphoreType.DMA,
        pltpu.SemaphoreType.DMA,
        pltpu.VMEM_SHARED((N_NODES, 64), F32),
    ],
)


BN = 2000
GN = N_NODES // BN


def _tc_node_body(accp, xin, wskip, wn1, bn1, wn2, bn2, wclin, wself,
                  selw, seld, hc_o, fs_o):
    acc = accp[0] + accp[1]
    attn_num = _mm(acc, selw[...])
    den = _mm(acc, seld[...])
    attn = attn_num / (den + 1e-9)
    h = attn + xin[...] @ wskip[...]
    nrm = jnp.abs(h)
    ph = h / (nrm + 1e-8)
    s1 = jnp.maximum(nrm @ wn1[...] + bn1[...], 0.0)
    s2 = jnp.maximum(s1 @ wn2[...] + bn2[...], 0.0)
    h2 = s2 * ph
    hc_o[...] = h2 @ wclin[...]
    fs_o[...] = h2 @ wself[...]


_tc_node = pl.pallas_call(
    _tc_node_body,
    grid=(GN,),
    in_specs=[
        pl.BlockSpec((NC, BN, 64), lambda i: (0, i, 0)),
        pl.BlockSpec((BN, 6), lambda i: (i, 0)),
        _full((6, 32)),
        _full((32, 32)), _full((1, 32)), _full((32, 32)), _full((1, 32)),
        _full((32, 128)), _full((32, 128)),
        _full((64, 32)), _full((64, 1)),
    ],
    out_specs=[
        pl.BlockSpec((BN, 128), lambda i: (i, 0)),
        pl.BlockSpec((BN, 128), lambda i: (i, 0)),
    ],
    out_shape=[
        jax.ShapeDtypeStruct((N_NODES, 128), F32),
        jax.ShapeDtypeStruct((N_NODES, 128), F32),
    ],
)


def _sc_conv_body(cmat, hcmat, srci, dsti, zrows, aggp_o,
                  sv, dv, cv, gv, agg, sem):
    cid = lax.axis_index("c")
    sid = lax.axis_index("s")
    wid = sid * NC + cid
    row0 = sid * ROWS_PT
    pltpu.sync_copy(zrows.at[pl.ds(row0, ROWS_PT)],
                    agg.at[pl.ds(row0, ROWS_PT)])
    plsc.subcore_barrier()

    @pl.loop(0, NCHUNK)
    def _chunk(i):
        base = wid * EPW + i * CH
        pltpu.sync_copy(srci.at[pl.ds(base, CH)], sv)
        pltpu.sync_copy(dsti.at[pl.ds(base, CH)], dv)
        cp = pltpu.async_copy(hcmat.at[sv], gv, sem)
        pltpu.sync_copy(cmat.at[pl.ds(base, CH)], cv)
        cp.wait()

        @pl.loop(0, CH)
        def _row(rr):
            for j in range(8):
                o = j * 16
                gv[rr, pl.ds(o, 16)] = (
                    gv[rr, pl.ds(o, 16)] * cv[rr, pl.ds(o, 16)])

        pltpu.sync_copy(gv, agg.at[dv], add=True)

    plsc.subcore_barrier()
    pltpu.sync_copy(agg.at[pl.ds(row0, ROWS_PT)],
                    aggp_o.at[cid, pl.ds(row0, ROWS_PT)])


_sc_conv = pl.kernel(
    _sc_conv_body,
    out_type=jax.ShapeDtypeStruct((NC, N_NODES, 128), F32),
    mesh=_sc_mesh(),
    compiler_params=pltpu.CompilerParams(use_tc_tiling_on_sc=False),
    scratch_types=[
        pltpu.VMEM((CH,), jnp.int32),
        pltpu.VMEM((CH,), jnp.int32),
        pltpu.VMEM((CH, 128), F32),
        pltpu.VMEM((CH, 128), F32),
        pltpu.VMEM_SHARED((N_NODES, 128), F32),
        pltpu.SemaphoreType.DMA,
    ],
)


def _tc_merge_body(aggp, fs, out_o):
    out_o[...] = aggp[0] + aggp[1] + fs[...]


_tc_merge = pl.pallas_call(
    _tc_merge_body,
    grid=(GN,),
    in_specs=[
        pl.BlockSpec((NC, BN, 128), lambda i: (0, i, 0)),
        pl.BlockSpec((BN, 128), lambda i: (i, 0)),
    ],
    out_specs=pl.BlockSpec((BN, 128), lambda i: (i, 0)),
    out_shape=jax.ShapeDtypeStruct((N_NODES, 128), F32),
)


def kernel(x, pos, edge_index, edge_attr,
           Wk1, bk1, Wk2, bk2, Wk_lin,
           Wv1, bv1, Wv2, bv2, Wv_lin,
           Wq, Wskip,
           Wn1, bn1, Wn2, bn2,
           Wc1, bc1, Wc2, bc2, Wc_lin, Wself):
    src = edge_index[0]
    dst = edge_index[1]
    npack = jnp.concatenate([pos, x, jnp.zeros((N_NODES, 7), F32)], axis=1)

    sg, dg = _sc_gather(npack, src, dst)

    def packed(w):
        return jnp.zeros((16, w.shape[1]), F32).at[3:9, :].set(w)

    def pad16(w):
        return jnp.concatenate([w, jnp.zeros((3, w.shape[1]), F32)], axis=0)

    z = jnp.zeros
    w1cat = jnp.concatenate([pad16(Wk1), pad16(Wv1), pad16(Wc1)], axis=1)
    b1cat = jnp.concatenate([bk1, bv1, bc1])[None]
    w2kv = jnp.concatenate([
        jnp.concatenate([Wk2, z((64, 32), F32)], axis=1),
        jnp.concatenate([z((64, 32), F32), Wv2], axis=1),
        z((128, 64), F32),
    ], axis=0)
    b2kv = jnp.concatenate([bk2, bv2])[None]
    wc2p = jnp.concatenate([z((128, 128), F32), Wc2], axis=0)
    wklvl = jnp.concatenate([packed(Wk_lin), packed(Wv_lin)], axis=1)

    selw = jnp.zeros((64, 32), F32).at[jnp.arange(32), jnp.arange(32)].set(1.0)
    seld = jnp.zeros((64, 1), F32).at[32, 0].set(1.0)

    w64, c = _tc_edge(
        sg, dg, edge_attr,
        w1cat, b1cat, w2kv, b2kv, wc2p, bc2[None], wklvl, packed(Wq),
    )

    z64 = jnp.zeros((N_NODES, 64), F32)
    accp = _sc_attn(w64, dst, z64)

    hc, fs = _tc_node(
        accp, x, Wskip, Wn1, bn1[None], Wn2, bn2[None], Wc_lin, Wself,
        selw, seld,
    )

    z128 = jnp.zeros((N_NODES, 128), F32)
    aggp = _sc_conv(c, hc, src, dst, z128)

    feat = _tc_merge(aggp, fs)
    return (feat[None], pos)


# speedup vs baseline: 5.0374x; 1.0123x over previous
import functools
import math

import jax
import jax.numpy as jnp
from jax import lax
from jax.experimental import pallas as pl
from jax.experimental.pallas import tpu as pltpu
from jax.experimental.pallas import tpu_sc as plsc

N_NODES = 10000
E_EDGES = 320000
NC, NS = 2, 16
NW = NC * NS
EPW = E_EDGES // NW
CH = 80
NCHUNK = EPW // CH
ROWS_PT = N_NODES // NS

F32 = jnp.float32


def _sc_mesh():
    return plsc.VectorSubcoreMesh(
        core_axis_name="c", subcore_axis_name="s", num_cores=NC, num_subcores=NS
    )


def _sc_gather_body(npack, srci, dsti, sg_o, dg_o,
                    siv0, div0, srows0, drows0, siv1, div1, srows1, drows1,
                    sa0, sb0, sa1, sb1):
    wid = lax.axis_index("s") * NC + lax.axis_index("c")
    bufs = ((siv0, div0, srows0, drows0, sa0, sb0),
            (siv1, div1, srows1, drows1, sa1, sb1))

    def start(i, b):
        siv, div, srows, drows, sa, sb = bufs[b]
        base = wid * EPW + i * CH
        pltpu.sync_copy(srci.at[pl.ds(base, CH)], siv)
        pltpu.sync_copy(dsti.at[pl.ds(base, CH)], div)
        pltpu.async_copy(npack.at[siv], srows, sa)
        pltpu.async_copy(npack.at[div], drows, sb)

    def finish(i, b):
        siv, div, srows, drows, sa, sb = bufs[b]
        base = wid * EPW + i * CH
        pltpu.make_async_copy(npack.at[siv], srows, sa).wait()
        pltpu.make_async_copy(npack.at[div], drows, sb).wait()
        pltpu.sync_copy(srows, sg_o.at[pl.ds(base, CH)])
        pltpu.sync_copy(drows, dg_o.at[pl.ds(base, CH)])

    start(0, 0)

    @pl.loop(0, (NCHUNK - 1) // 2)
    def _pair(p):
        i0 = 2 * p
        start(i0 + 1, 1)
        finish(i0, 0)
        start(i0 + 2, 0)
        finish(i0 + 1, 1)

    finish(NCHUNK - 1, 0)


_sc_gather = pl.kernel(
    _sc_gather_body,
    out_type=(
        jax.ShapeDtypeStruct((E_EDGES, 16), F32),
        jax.ShapeDtypeStruct((E_EDGES, 16), F32),
    ),
    mesh=_sc_mesh(),
    compiler_params=pltpu.CompilerParams(use_tc_tiling_on_sc=False),
    scratch_types=[
        pltpu.VMEM((CH,), jnp.int32),
        pltpu.VMEM((CH,), jnp.int32),
        pltpu.VMEM((CH, 16), F32),
        pltpu.VMEM((CH, 16), F32),
        pltpu.VMEM((CH,), jnp.int32),
        pltpu.VMEM((CH,), jnp.int32),
        pltpu.VMEM((CH, 16), F32),
        pltpu.VMEM((CH, 16), F32),
        pltpu.SemaphoreType.DMA,
        pltpu.SemaphoreType.DMA,
        pltpu.SemaphoreType.DMA,
        pltpu.SemaphoreType.DMA,
    ],
)


BE = 2000
GE = E_EDGES // BE

_RSQRT_F = float(1.0 / math.sqrt(32.0))

_mm = functools.partial(jnp.matmul, precision=jax.lax.Precision.HIGHEST)


def _tc_edge_body(sg, dg, attr, wk1p, bk1, wk2, bk2, wklp,
                  wv1p, bv1, wv2, bv2, wvlp, wqp,
                  wc1p, bc1, wc2, bc2, w64_o, c_o):
    sgv = sg[...]
    dgv = dg[...]
    d = dgv - sgv
    d3 = d[:, 0:3]
    r2 = jnp.sum(d3 * d3, axis=1, keepdims=True) + 1e-8
    r = jnp.sqrt(r2)
    at = attr[...]
    ein = jnp.concatenate([r, at, jnp.zeros((BE, 3), F32)], axis=1)
    hk = jnp.maximum(ein @ wk1p[...] + bk1[...], 0.0)
    krad = hk @ wk2[...] + bk2[...]
    kl = sgv @ wklp[...]
    kq = dgv @ wqp[...]
    logits = jnp.sum(krad * kl * kq, axis=1, keepdims=True) * _RSQRT_F
    ex = jnp.exp(logits)
    hv = jnp.maximum(ein @ wv1p[...] + bv1[...], 0.0)
    vrad = hv @ wv2[...] + bv2[...]
    vl = sgv @ wvlp[...]
    w = ex * (vrad * vl)
    w64_o[...] = jnp.concatenate([w, ex, jnp.zeros((BE, 31), F32)], axis=1)
    hc = jnp.maximum(ein @ wc1p[...] + bc1[...], 0.0)
    c_o[...] = hc @ wc2[...] + bc2[...]


def _full(shape):
    rank = len(shape)
    return pl.BlockSpec(shape, lambda i, _r=rank: (0,) * _r)


_tc_edge = pl.pallas_call(
    _tc_edge_body,
    grid=(GE,),
    in_specs=[
        pl.BlockSpec((BE, 16), lambda i: (i, 0)),
        pl.BlockSpec((BE, 16), lambda i: (i, 0)),
        pl.BlockSpec((BE, 12), lambda i: (i, 0)),
        _full((16, 64)), _full((1, 64)),
        _full((64, 32)), _full((1, 32)), _full((16, 32)),
        _full((16, 64)), _full((1, 64)),
        _full((64, 32)), _full((1, 32)), _full((16, 32)),
        _full((16, 32)),
        _full((16, 64)), _full((1, 64)),
        _full((64, 128)), _full((1, 128)),
    ],
    out_specs=[
        pl.BlockSpec((BE, 64), lambda i: (i, 0)),
        pl.BlockSpec((BE, 128), lambda i: (i, 0)),
    ],
    out_shape=[
        jax.ShapeDtypeStruct((E_EDGES, 64), F32),
        jax.ShapeDtypeStruct((E_EDGES, 128), F32),
    ],
)


def _sc_attn_body(w64, dsti, zrows, accp_o, dv0, wv0, dv1, wv1, sa0, sa1,
                  acc):
    cid = lax.axis_index("c")
    sid = lax.axis_index("s")
    wid = sid * NC + cid
    row0 = sid * ROWS_PT
    pltpu.sync_copy(zrows.at[pl.ds(row0, ROWS_PT)],
                    acc.at[pl.ds(row0, ROWS_PT)])
    bufs = ((dv0, wv0, sa0), (dv1, wv1, sa1))

    def start(i, b):
        dv, wv, sa = bufs[b]
        base = wid * EPW + i * CH
        pltpu.sync_copy(dsti.at[pl.ds(base, CH)], dv)
        pltpu.async_copy(w64.at[pl.ds(base, CH)], wv, sa)

    def finish(i, b):
        dv, wv, sa = bufs[b]
        base = wid * EPW + i * CH
        pltpu.make_async_copy(w64.at[pl.ds(base, CH)], wv, sa).wait()
        pltpu.sync_copy(wv, acc.at[dv], add=True)

    plsc.subcore_barrier()
    start(0, 0)

    @pl.loop(0, (NCHUNK - 1) // 2)
    def _pair(p):
        i0 = 2 * p
        start(i0 + 1, 1)
        finish(i0, 0)
        start(i0 + 2, 0)
        finish(i0 + 1, 1)

    finish(NCHUNK - 1, 0)
    plsc.subcore_barrier()
    pltpu.sync_copy(acc.at[pl.ds(row0, ROWS_PT)],
                    accp_o.at[cid, pl.ds(row0, ROWS_PT)])


_sc_attn = pl.kernel(
    _sc_attn_body,
    out_type=jax.ShapeDtypeStruct((NC, N_NODES, 64), F32),
    mesh=_sc_mesh(),
    compiler_params=pltpu.CompilerParams(use_tc_tiling_on_sc=False),
    scratch_types=[
        pltpu.VMEM((CH,), jnp.int32),
        pltpu.VMEM((CH, 64), F32),
        pltpu.VMEM((CH,), jnp.int32),
        pltpu.VMEM((CH, 64), F32),
        pltpu.SemaphoreType.DMA,
        pltpu.SemaphoreType.DMA,
        pltpu.VMEM_SHARED((N_NODES, 64), F32),
    ],
)


BN = 2000
GN = N_NODES // BN


def _tc_node_body(accp, xin, wskip, wn1, bn1, wn2, bn2, wclin, wself,
                  selw, seld, hc_o, fs_o):
    acc = accp[0] + accp[1]
    attn_num = _mm(acc, selw[...])
    den = _mm(acc, seld[...])
    attn = attn_num / (den + 1e-9)
    h = attn + xin[...] @ wskip[...]
    nrm = jnp.abs(h)
    ph = h / (nrm + 1e-8)
    s1 = jnp.maximum(nrm @ wn1[...] + bn1[...], 0.0)
    s2 = jnp.maximum(s1 @ wn2[...] + bn2[...], 0.0)
    h2 = s2 * ph
    hc_o[...] = h2 @ wclin[...]
    fs_o[...] = h2 @ wself[...]


_tc_node = pl.pallas_call(
    _tc_node_body,
    grid=(GN,),
    in_specs=[
        pl.BlockSpec((NC, BN, 64), lambda i: (0, i, 0)),
        pl.BlockSpec((BN, 6), lambda i: (i, 0)),
        _full((6, 32)),
        _full((32, 32)), _full((1, 32)), _full((32, 32)), _full((1, 32)),
        _full((32, 128)), _full((32, 128)),
        _full((64, 32)), _full((64, 1)),
    ],
    out_specs=[
        pl.BlockSpec((BN, 128), lambda i: (i, 0)),
        pl.BlockSpec((BN, 128), lambda i: (i, 0)),
    ],
    out_shape=[
        jax.ShapeDtypeStruct((N_NODES, 128), F32),
        jax.ShapeDtypeStruct((N_NODES, 128), F32),
    ],
)


def _sc_conv_body(cmat, hcmat, srci, dsti, zrows, aggp_o,
                  sv, dv, cv, gv, agg, sem):
    cid = lax.axis_index("c")
    sid = lax.axis_index("s")
    wid = sid * NC + cid
    row0 = sid * ROWS_PT
    pltpu.sync_copy(zrows.at[pl.ds(row0, ROWS_PT)],
                    agg.at[pl.ds(row0, ROWS_PT)])
    plsc.subcore_barrier()

    @pl.loop(0, NCHUNK)
    def _chunk(i):
        base = wid * EPW + i * CH
        pltpu.sync_copy(srci.at[pl.ds(base, CH)], sv)
        pltpu.sync_copy(dsti.at[pl.ds(base, CH)], dv)
        cp = pltpu.async_copy(hcmat.at[sv], gv, sem)
        pltpu.sync_copy(cmat.at[pl.ds(base, CH)], cv)
        cp.wait()

        @pl.loop(0, CH)
        def _row(rr):
            for j in range(8):
                o = j * 16
                gv[rr, pl.ds(o, 16)] = (
                    gv[rr, pl.ds(o, 16)] * cv[rr, pl.ds(o, 16)])

        pltpu.sync_copy(gv, agg.at[dv], add=True)

    plsc.subcore_barrier()
    pltpu.sync_copy(agg.at[pl.ds(row0, ROWS_PT)],
                    aggp_o.at[cid, pl.ds(row0, ROWS_PT)])


_sc_conv = pl.kernel(
    _sc_conv_body,
    out_type=jax.ShapeDtypeStruct((NC, N_NODES, 128), F32),
    mesh=_sc_mesh(),
    compiler_params=pltpu.CompilerParams(use_tc_tiling_on_sc=False),
    scratch_types=[
        pltpu.VMEM((CH,), jnp.int32),
        pltpu.VMEM((CH,), jnp.int32),
        pltpu.VMEM((CH, 128), F32),
        pltpu.VMEM((CH, 128), F32),
        pltpu.VMEM_SHARED((N_NODES, 128), F32),
        pltpu.SemaphoreType.DMA,
    ],
)


def _tc_merge_body(aggp, fs, out_o):
    out_o[...] = aggp[0] + aggp[1] + fs[...]


_tc_merge = pl.pallas_call(
    _tc_merge_body,
    grid=(GN,),
    in_specs=[
        pl.BlockSpec((NC, BN, 128), lambda i: (0, i, 0)),
        pl.BlockSpec((BN, 128), lambda i: (i, 0)),
    ],
    out_specs=pl.BlockSpec((BN, 128), lambda i: (i, 0)),
    out_shape=jax.ShapeDtypeStruct((N_NODES, 128), F32),
)


def kernel(x, pos, edge_index, edge_attr,
           Wk1, bk1, Wk2, bk2, Wk_lin,
           Wv1, bv1, Wv2, bv2, Wv_lin,
           Wq, Wskip,
           Wn1, bn1, Wn2, bn2,
           Wc1, bc1, Wc2, bc2, Wc_lin, Wself):
    src = edge_index[0]
    dst = edge_index[1]
    npack = jnp.concatenate([pos, x, jnp.zeros((N_NODES, 7), F32)], axis=1)

    sg, dg = _sc_gather(npack, src, dst)

    def packed(w):
        return jnp.zeros((16, w.shape[1]), F32).at[3:9, :].set(w)

    def pad16(w):
        return jnp.concatenate([w, jnp.zeros((3, w.shape[1]), F32)], axis=0)

    selw = jnp.zeros((64, 32), F32).at[jnp.arange(32), jnp.arange(32)].set(1.0)
    seld = jnp.zeros((64, 1), F32).at[32, 0].set(1.0)

    w64, c = _tc_edge(
        sg, dg, edge_attr,
        pad16(Wk1), bk1[None], Wk2, bk2[None], packed(Wk_lin),
        pad16(Wv1), bv1[None], Wv2, bv2[None], packed(Wv_lin),
        packed(Wq),
        pad16(Wc1), bc1[None], Wc2, bc2[None],
    )

    z64 = jnp.zeros((N_NODES, 64), F32)
    accp = _sc_attn(w64, dst, z64)

    hc, fs = _tc_node(
        accp, x, Wskip, Wn1, bn1[None], Wn2, bn2[None], Wc_lin, Wself,
        selw, seld,
    )

    z128 = jnp.zeros((N_NODES, 128), F32)
    aggp = _sc_conv(c, hc, src, dst, z128)

    feat = _tc_merge(aggp, fs)
    return (feat[None], pos)
